# speedup vs baseline: 3.2672x; 3.2672x over previous
import functools

import jax
import jax.numpy as jnp
from jax import lax
from jax.experimental import pallas as pl
from jax.experimental.pallas import tpu as pltpu
from jax.experimental.pallas import tpu_sc as plsc

NUM_EMB = 100000
DIM = 128
BATCH = 4096 * 50
NUM_CORES = 2
NUM_SUBCORES = 16
NUM_WORKERS = NUM_CORES * NUM_SUBCORES
B_PER_W = BATCH // NUM_WORKERS
CHUNK = 640
N_CHUNKS = B_PER_W // CHUNK


def _emb_body(table_hbm, idx_hbm, out_hbm, idx_v, rows_v, sem):
    wid = lax.axis_index("s") * NUM_CORES + lax.axis_index("c")
    base = wid * B_PER_W

    def body(i, _):
        off = base + i * CHUNK
        pltpu.sync_copy(idx_hbm.at[pl.ds(off, CHUNK)], idx_v)
        pltpu.async_copy(table_hbm.at[idx_v], rows_v, sem).wait()
        pltpu.sync_copy(rows_v, out_hbm.at[pl.ds(off, CHUNK)])
        return 0

    lax.fori_loop(0, N_CHUNKS, body, 0)


@jax.jit
def _embed(table, idx):
    mesh = plsc.VectorSubcoreMesh(core_axis_name="c", subcore_axis_name="s")
    return pl.kernel(
        _emb_body,
        mesh=mesh,
        out_type=jax.ShapeDtypeStruct((BATCH, DIM), jnp.float32),
        scratch_types=[
            pltpu.VMEM((CHUNK,), jnp.int32),
            pltpu.VMEM((CHUNK, DIM), jnp.float32),
            pltpu.SemaphoreType.DMA,
        ],
    )(table, idx)


def kernel(token_ids, embedding_matrix):
    idx = token_ids.reshape(-1).astype(jnp.int32)
    out = _embed(embedding_matrix, idx)
    return out.reshape(token_ids.shape + (DIM,))


# speedup vs baseline: 3.3107x; 1.0133x over previous
import jax
import jax.numpy as jnp
from jax import lax
from jax.experimental import pallas as pl
from jax.experimental.pallas import tpu as pltpu
from jax.experimental.pallas import tpu_sc as plsc

NUM_EMB = 100000
DIM = 128
BATCH = 4096 * 50
NUM_CORES = 2
NUM_SUBCORES = 16
NUM_WORKERS = NUM_CORES * NUM_SUBCORES
B_PER_W = BATCH // NUM_WORKERS
CHUNK = 400
N_CHUNKS = B_PER_W // CHUNK


def _emb_body(table_hbm, idx_hbm, out_hbm,
              idx_all, rows0, rows1, g0, g1, s0, s1):
    wid = lax.axis_index("s") * NUM_CORES + lax.axis_index("c")
    base = wid * B_PER_W

    rows = (rows0, rows1)
    gsem = (g0, g1)
    ssem = (s0, s1)

    pltpu.sync_copy(idx_hbm.at[pl.ds(base, B_PER_W)], idx_all)

    def gather(i, b):
        return pltpu.async_copy(
            table_hbm.at[idx_all.at[pl.ds(i * CHUNK, CHUNK)]], rows[b], gsem[b])

    gath = gather(0, 0)
    scat = [None, None]
    for i in range(N_CHUNKS):
        b = i % 2
        nb = 1 - b
        gath.wait()
        scat[b] = pltpu.async_copy(
            rows[b], out_hbm.at[pl.ds(base + i * CHUNK, CHUNK)], ssem[b])
        if i + 1 < N_CHUNKS:
            if scat[nb] is not None:
                scat[nb].wait()
            gath = gather(i + 1, nb)
    scat[0].wait()
    scat[1].wait()


@jax.jit
def _embed(table, idx):
    mesh = plsc.VectorSubcoreMesh(core_axis_name="c", subcore_axis_name="s")
    return pl.kernel(
        _emb_body,
        mesh=mesh,
        out_type=jax.ShapeDtypeStruct((BATCH, DIM), jnp.float32),
        scratch_types=[
            pltpu.VMEM((B_PER_W,), jnp.int32),
            pltpu.VMEM((CHUNK, DIM), jnp.float32),
            pltpu.VMEM((CHUNK, DIM), jnp.float32),
            pltpu.SemaphoreType.DMA,
            pltpu.SemaphoreType.DMA,
            pltpu.SemaphoreType.DMA,
            pltpu.SemaphoreType.DMA,
        ],
    )(table, idx)


def kernel(token_ids, embedding_matrix):
    idx = token_ids.reshape(-1).astype(jnp.int32)
    out = _embed(embedding_matrix, idx)
    return out.reshape(token_ids.shape + (DIM,))


# speedup vs baseline: 5.7739x; 1.7440x over previous
import jax
import jax.numpy as jnp
from jax import lax
from jax.experimental import pallas as pl
from jax.experimental.pallas import tpu as pltpu
from jax.experimental.pallas import tpu_sc as plsc

NUM_EMB = 100000
DIM = 128
SEQS = 4096
SEQ_LEN = 50
BATCH = SEQS * SEQ_LEN
NUM_CORES = 2
NUM_SUBCORES = 16
NUM_WORKERS = NUM_CORES * NUM_SUBCORES
B_PER_W = BATCH // NUM_WORKERS
SEQ_PER_W = SEQS // NUM_WORKERS
SEQ_CHUNK = 8
CHUNK = SEQ_CHUNK * SEQ_LEN
N_CHUNKS = B_PER_W // CHUNK


def _emb_body(table_hbm, idx_hbm, out_hbm,
              idx_all, rows0, rows1, g0, g1, s0, s1):
    wid = lax.axis_index("s") * NUM_CORES + lax.axis_index("c")
    base = wid * B_PER_W
    seq_base = wid * SEQ_PER_W

    rows = (rows0, rows1)
    gsem = (g0, g1)
    ssem = (s0, s1)

    pltpu.sync_copy(idx_hbm.at[pl.ds(base, B_PER_W)], idx_all)

    def gather(i, b):
        return pltpu.async_copy(
            table_hbm.at[idx_all.at[pl.ds(i * CHUNK, CHUNK)]], rows[b], gsem[b])

    def scatter(i, b):
        descs = []
        for s in range(SEQ_CHUNK):
            descs.append(pltpu.async_copy(
                rows[b].at[pl.ds(s * SEQ_LEN, SEQ_LEN)],
                out_hbm.at[seq_base + i * SEQ_CHUNK + s],
                ssem[b]))
        return descs

    gath = gather(0, 0)
    scat = [None, None]
    for i in range(N_CHUNKS):
        b = i % 2
        nb = 1 - b
        gath.wait()
        scat[b] = scatter(i, b)
        if i + 1 < N_CHUNKS:
            if scat[nb] is not None:
                for d in scat[nb]:
                    d.wait()
            gath = gather(i + 1, nb)
    for b in (0, 1):
        for d in scat[b]:
            d.wait()


@jax.jit
def _embed(table, idx):
    mesh = plsc.VectorSubcoreMesh(core_axis_name="c", subcore_axis_name="s")
    return pl.kernel(
        _emb_body,
        mesh=mesh,
        out_type=jax.ShapeDtypeStruct((SEQS, SEQ_LEN, DIM), jnp.float32),
        scratch_types=[
            pltpu.VMEM((B_PER_W,), jnp.int32),
            pltpu.VMEM((CHUNK, DIM), jnp.float32),
            pltpu.VMEM((CHUNK, DIM), jnp.float32),
            pltpu.SemaphoreType.DMA,
            pltpu.SemaphoreType.DMA,
            pltpu.SemaphoreType.DMA,
            pltpu.SemaphoreType.DMA,
        ],
    )(table, idx)


def kernel(token_ids, embedding_matrix):
    idx = token_ids.reshape(-1).astype(jnp.int32)
    return _embed(embedding_matrix, idx)


# speedup vs baseline: 5.7750x; 1.0002x over previous
import jax
import jax.numpy as jnp
from jax import lax
from jax.experimental import pallas as pl
from jax.experimental.pallas import tpu as pltpu
from jax.experimental.pallas import tpu_sc as plsc

NUM_EMB = 100000
DIM = 128
SEQS = 4096
SEQ_LEN = 50
BATCH = SEQS * SEQ_LEN
NUM_CORES = 2
NUM_SUBCORES = 16
NUM_WORKERS = NUM_CORES * NUM_SUBCORES
B_PER_W = BATCH // NUM_WORKERS
SEQ_PER_W = SEQS // NUM_WORKERS
SEQ_CHUNK = 8
CHUNK = SEQ_CHUNK * SEQ_LEN
N_CHUNKS = B_PER_W // CHUNK


def _emb_body(table_hbm, idx_hbm, out_hbm,
              idx_all, rows0, rows1, g0, g1, s0, s1):
    wid = lax.axis_index("s") * NUM_CORES + lax.axis_index("c")
    base = wid * B_PER_W
    seq_base = wid * SEQ_PER_W

    rows = (rows0, rows1)
    gsem = (g0, g1)
    ssem = (s0, s1)

    pltpu.sync_copy(idx_hbm.at[pl.ds(base, B_PER_W)], idx_all)

    def gather(i, b):
        return pltpu.async_copy(
            table_hbm.at[idx_all.at[pl.ds(i * CHUNK, CHUNK)]], rows[b], gsem[b])

    def scatter(i, b):
        descs = []
        for s in range(SEQ_CHUNK):
            descs.append(pltpu.async_copy(
                rows[b].at[pl.ds(s * SEQ_LEN, SEQ_LEN)],
                out_hbm.at[seq_base + i * SEQ_CHUNK + s],
                ssem[b]))
        return descs

    gath = gather(0, 0)
    scat = [None, None]
    for i in range(N_CHUNKS):
        b = i % 2
        nb = 1 - b
        gath.wait()
        scat[b] = scatter(i, b)
        if i + 1 < N_CHUNKS:
            if scat[nb] is not None:
                for d in scat[nb]:
                    d.wait()
            gath = gather(i + 1, nb)
    for b in (0, 1):
        for d in scat[b]:
            d.wait()


@jax.jit
def _embed(table, idx):
    mesh = plsc.VectorSubcoreMesh(core_axis_name="c", subcore_axis_name="s")
    return pl.kernel(
        _emb_body,
        mesh=mesh,
        compiler_params=pltpu.CompilerParams(use_tc_tiling_on_sc=True),
        out_type=jax.ShapeDtypeStruct((SEQS, SEQ_LEN, DIM), jnp.float32),
        scratch_types=[
            pltpu.VMEM((B_PER_W,), jnp.int32),
            pltpu.VMEM((CHUNK, DIM), jnp.float32),
            pltpu.VMEM((CHUNK, DIM), jnp.float32),
            pltpu.SemaphoreType.DMA,
            pltpu.SemaphoreType.DMA,
            pltpu.SemaphoreType.DMA,
            pltpu.SemaphoreType.DMA,
        ],
    )(table, idx)


def kernel(token_ids, embedding_matrix):
    idx = token_ids.reshape(-1).astype(jnp.int32)
    return _embed(embedding_matrix, idx)


# speedup vs baseline: 10.4033x; 1.8014x over previous
import jax
import jax.numpy as jnp
from jax import lax
from jax.experimental import pallas as pl
from jax.experimental.pallas import tpu as pltpu
from jax.experimental.pallas import tpu_sc as plsc

NUM_EMB = 100000
DIM = 128
SEQS = 4096
SEQ_LEN = 50
BATCH = SEQS * SEQ_LEN
NUM_CORES = 2
NUM_SUBCORES = 16
NUM_WORKERS = NUM_CORES * NUM_SUBCORES
B_PER_W = BATCH // NUM_WORKERS
CHUNK = 400
N_CHUNKS = B_PER_W // CHUNK


def _emb_body(table_hbm, idx_hbm, out_hbm,
              idx_all, rows0, rows1, g0, g1, s0, s1):
    wid = lax.axis_index("s") * NUM_CORES + lax.axis_index("c")
    base = wid * B_PER_W

    rows = (rows0, rows1)
    gsem = (g0, g1)
    ssem = (s0, s1)

    pltpu.sync_copy(idx_hbm.at[pl.ds(base, B_PER_W)], idx_all)

    def gather(i, b):
        return pltpu.async_copy(
            table_hbm.at[idx_all.at[pl.ds(i * CHUNK, CHUNK)]], rows[b], gsem[b])

    gath = gather(0, 0)
    scat = [None, None]
    for i in range(N_CHUNKS):
        b = i % 2
        nb = 1 - b
        gath.wait()
        scat[b] = pltpu.async_copy(
            rows[b], out_hbm.at[pl.ds(base + i * CHUNK, CHUNK)], ssem[b])
        if i + 1 < N_CHUNKS:
            if scat[nb] is not None:
                scat[nb].wait()
            gath = gather(i + 1, nb)
    scat[0].wait()
    scat[1].wait()


@jax.jit
def _embed(table, idx):
    mesh = plsc.VectorSubcoreMesh(core_axis_name="c", subcore_axis_name="s")
    return pl.kernel(
        _emb_body,
        mesh=mesh,
        out_type=jax.ShapeDtypeStruct((BATCH, DIM), jnp.float32),
        scratch_types=[
            pltpu.VMEM((B_PER_W,), jnp.int32),
            pltpu.VMEM((CHUNK, DIM), jnp.float32),
            pltpu.VMEM((CHUNK, DIM), jnp.float32),
            pltpu.SemaphoreType.DMA,
            pltpu.SemaphoreType.DMA,
            pltpu.SemaphoreType.DMA,
            pltpu.SemaphoreType.DMA,
        ],
    )(table, idx)


def kernel(token_ids, embedding_matrix):
    idx = token_ids.T.reshape(-1).astype(jnp.int32)
    out = _embed(embedding_matrix, idx)
    return out.reshape(SEQ_LEN, SEQS, DIM).swapaxes(0, 1)
